# Initial kernel scaffold; baseline (speedup 1.0000x reference)
import jax
import jax.numpy as jnp
from jax import lax
from jax.experimental import pallas as pl
from jax.experimental.pallas import tpu as pltpu
from jax.experimental.pallas import tpu_sc as plsc

N_COLS = 26
SUB_DIM = 32
BATCH = 16384
ENTITY_DIM = N_COLS * SUB_DIM
HIDDEN = 256
OUT_DIM = 16

_NC = 2
_NS = 16
_NW = _NC * _NS
_BPW = BATCH // _NW


def _gather_body(*refs):
    cols = refs[:N_COLS]
    tables = refs[N_COLS:2 * N_COLS]
    out = refs[2 * N_COLS]
    idx_v, rows_v, sem = refs[2 * N_COLS + 1:]
    wid = lax.axis_index("s") * _NC + lax.axis_index("c")
    base = wid * _BPW
    for i in range(N_COLS):
        pltpu.sync_copy(cols[i].at[pl.ds(base, _BPW)], idx_v)
        pltpu.async_copy(tables[i].at[idx_v], rows_v, sem).wait()
        pltpu.sync_copy(
            rows_v, out.at[pl.ds(base, _BPW), pl.ds(i * SUB_DIM, SUB_DIM)]
        )


@jax.jit
def _gather_entity(cols, tables):
    mesh = plsc.VectorSubcoreMesh(core_axis_name="c", subcore_axis_name="s")
    k = pl.kernel(
        _gather_body,
        mesh=mesh,
        out_type=jax.ShapeDtypeStruct((BATCH, ENTITY_DIM), jnp.float32),
        scratch_types=[
            pltpu.VMEM((_BPW,), jnp.int32),
            pltpu.VMEM((_BPW, SUB_DIM), jnp.float32),
            pltpu.SemaphoreType.DMA,
        ],
    )
    return k(*cols, *tables)


def _mlp_body(e_ref, w1_ref, b1_ref, w2_ref, b2_ref, o_ref):
    h = jnp.dot(e_ref[...], w1_ref[...], preferred_element_type=jnp.float32)
    h = jnp.maximum(h + b1_ref[...], 0.0)
    o = jnp.dot(h, w2_ref[...], preferred_element_type=jnp.float32)
    o_ref[...] = o + b2_ref[...]


@jax.jit
def _mlp(entity, W1, b1, W2, b2):
    BM = 2048
    grid = (BATCH // BM,)
    return pl.pallas_call(
        _mlp_body,
        grid=grid,
        in_specs=[
            pl.BlockSpec((BM, ENTITY_DIM), lambda i: (i, 0)),
            pl.BlockSpec((ENTITY_DIM, HIDDEN), lambda i: (0, 0)),
            pl.BlockSpec((1, HIDDEN), lambda i: (0, 0)),
            pl.BlockSpec((HIDDEN, OUT_DIM), lambda i: (0, 0)),
            pl.BlockSpec((1, OUT_DIM), lambda i: (0, 0)),
        ],
        out_specs=pl.BlockSpec((BM, OUT_DIM), lambda i: (i, 0)),
        out_shape=jax.ShapeDtypeStruct((BATCH, OUT_DIM), jnp.float32),
    )(entity, W1, b1.reshape(1, HIDDEN), W2, b2.reshape(1, OUT_DIM))


def kernel(col_0, col_1, col_2, col_3, col_4, col_5, col_6, col_7, col_8, col_9, col_10, col_11, col_12, col_13, col_14, col_15, col_16, col_17, col_18, col_19, col_20, col_21, col_22, col_23, col_24, col_25, table_0, table_1, table_2, table_3, table_4, table_5, table_6, table_7, table_8, table_9, table_10, table_11, table_12, table_13, table_14, table_15, table_16, table_17, table_18, table_19, table_20, table_21, table_22, table_23, table_24, table_25, W1, b1, W2, b2):
    cols = [col_0, col_1, col_2, col_3, col_4, col_5, col_6, col_7, col_8,
            col_9, col_10, col_11, col_12, col_13, col_14, col_15, col_16,
            col_17, col_18, col_19, col_20, col_21, col_22, col_23, col_24,
            col_25]
    tables = [table_0, table_1, table_2, table_3, table_4, table_5, table_6,
              table_7, table_8, table_9, table_10, table_11, table_12,
              table_13, table_14, table_15, table_16, table_17, table_18,
              table_19, table_20, table_21, table_22, table_23, table_24,
              table_25]
    entity = _gather_entity(cols, tables)
    return _mlp(entity, W1, b1, W2, b2)


# speedup vs baseline: 1.8731x; 1.8731x over previous
import jax
import jax.numpy as jnp
from jax import lax
from jax.experimental import pallas as pl
from jax.experimental.pallas import tpu as pltpu
from jax.experimental.pallas import tpu_sc as plsc

N_COLS = 26
SUB_DIM = 32
BATCH = 16384
HIDDEN = 256
OUT_DIM = 16

_NC = 2
_NS = 16
_NW = _NC * _NS
_BPW = BATCH // _NW


def _gather_body(*refs):
    cols = refs[:N_COLS]
    tables = refs[N_COLS:2 * N_COLS]
    out = refs[2 * N_COLS]
    idx_v, rows_v, sem = refs[2 * N_COLS + 1:]
    wid = lax.axis_index("s") * _NC + lax.axis_index("c")
    base = wid * _BPW
    for c in range(N_COLS):
        pltpu.sync_copy(cols[c].at[pl.ds(base, _BPW)], idx_v)
        pltpu.async_copy(tables[c].at[idx_v], rows_v, sem).wait()
        pltpu.sync_copy(rows_v, out.at[c, pl.ds(base, _BPW)])


@jax.jit
def _gather_entity(cols, tables):
    mesh = plsc.VectorSubcoreMesh(core_axis_name="c", subcore_axis_name="s")
    k = pl.kernel(
        _gather_body,
        mesh=mesh,
        compiler_params=pltpu.CompilerParams(use_tc_tiling_on_sc=False),
        out_type=jax.ShapeDtypeStruct((N_COLS, BATCH, SUB_DIM), jnp.float32),
        scratch_types=[
            pltpu.VMEM((_BPW,), jnp.int32),
            pltpu.VMEM((_BPW, SUB_DIM), jnp.float32),
            pltpu.SemaphoreType.DMA,
        ],
    )
    return k(*cols, *tables)


def _mlp_body(e_ref, w1_ref, b1_ref, w2_ref, b2_ref, o_ref):
    h = jnp.dot(e_ref[0], w1_ref[0], preferred_element_type=jnp.float32)
    for c in range(1, N_COLS):
        h = h + jnp.dot(e_ref[c], w1_ref[c], preferred_element_type=jnp.float32)
    h = jnp.maximum(h + b1_ref[...], 0.0)
    o = jnp.dot(h, w2_ref[...], preferred_element_type=jnp.float32)
    o_ref[...] = o + b2_ref[...]


@jax.jit
def _mlp(entity, W1, b1, W2, b2):
    BM = 1024
    grid = (BATCH // BM,)
    W1r = W1.reshape(N_COLS, SUB_DIM, HIDDEN)
    return pl.pallas_call(
        _mlp_body,
        grid=grid,
        in_specs=[
            pl.BlockSpec((N_COLS, BM, SUB_DIM), lambda i: (0, i, 0)),
            pl.BlockSpec((N_COLS, SUB_DIM, HIDDEN), lambda i: (0, 0, 0)),
            pl.BlockSpec((1, HIDDEN), lambda i: (0, 0)),
            pl.BlockSpec((HIDDEN, OUT_DIM), lambda i: (0, 0)),
            pl.BlockSpec((1, OUT_DIM), lambda i: (0, 0)),
        ],
        out_specs=pl.BlockSpec((BM, OUT_DIM), lambda i: (i, 0)),
        out_shape=jax.ShapeDtypeStruct((BATCH, OUT_DIM), jnp.float32),
    )(entity, W1r, b1.reshape(1, HIDDEN), W2, b2.reshape(1, OUT_DIM))


def kernel(col_0, col_1, col_2, col_3, col_4, col_5, col_6, col_7, col_8, col_9, col_10, col_11, col_12, col_13, col_14, col_15, col_16, col_17, col_18, col_19, col_20, col_21, col_22, col_23, col_24, col_25, table_0, table_1, table_2, table_3, table_4, table_5, table_6, table_7, table_8, table_9, table_10, table_11, table_12, table_13, table_14, table_15, table_16, table_17, table_18, table_19, table_20, table_21, table_22, table_23, table_24, table_25, W1, b1, W2, b2):
    cols = [col_0, col_1, col_2, col_3, col_4, col_5, col_6, col_7, col_8,
            col_9, col_10, col_11, col_12, col_13, col_14, col_15, col_16,
            col_17, col_18, col_19, col_20, col_21, col_22, col_23, col_24,
            col_25]
    tables = [table_0, table_1, table_2, table_3, table_4, table_5, table_6,
              table_7, table_8, table_9, table_10, table_11, table_12,
              table_13, table_14, table_15, table_16, table_17, table_18,
              table_19, table_20, table_21, table_22, table_23, table_24,
              table_25]
    entity = _gather_entity(cols, tables)
    return _mlp(entity, W1, b1, W2, b2)
